# Initial kernel scaffold; baseline (speedup 1.0000x reference)
import jax
import jax.numpy as jnp
from jax.experimental import pallas as pl


def kernel(face_x, context_x, scene_x, params, face_edge_index, context_edge_index, face_batch, context_batch):
    raise NotImplementedError("write your pallas kernel here")


# speedup vs baseline: 1.0904x; 1.0904x over previous
import functools
import jax
import jax.numpy as jnp
from jax.experimental import pallas as pl
from jax.experimental.pallas import tpu as pltpu

D = 512
H = 4
C = 128
_EPS = 1e-5


def _ln_in(x, g, b):
    m = jnp.mean(x, axis=-1, keepdims=True)
    v = jnp.mean((x - m) ** 2, axis=-1, keepdims=True)
    return (x - m) * jax.lax.rsqrt(v + _EPS) * g + b


def _dotT(x, w):
    return jax.lax.dot_general(x, w, (((1,), (1,)), ((), ())),
                               preferred_element_type=jnp.float32)


def _enc1_body(pre_ln, x_ref, g0_ref, b0_ref, w_ref, b_ref, g1_ref, b1_ref,
               o_ref):
    x = x_ref[...]
    if pre_ln:
        x = _ln_in(x, g0_ref[...], b0_ref[...])
    y = _dotT(x, w_ref[...]) + b_ref[...]
    y = _ln_in(y, g1_ref[...], b1_ref[...])
    o_ref[...] = jnp.maximum(y, 0.0)


def _enc1(x, g0, b0, w, b, g1, b1, pre_ln=True, br=512):
    R, K = x.shape
    O = w.shape[0]
    if R < br:
        br = R
    grid = (R // br,)
    return pl.pallas_call(
        functools.partial(_enc1_body, pre_ln),
        grid=grid,
        in_specs=[
            pl.BlockSpec((br, K), lambda i: (i, 0)),
            pl.BlockSpec((1, K), lambda i: (0, 0)),
            pl.BlockSpec((1, K), lambda i: (0, 0)),
            pl.BlockSpec((O, K), lambda i: (0, 0)),
            pl.BlockSpec((1, O), lambda i: (0, 0)),
            pl.BlockSpec((1, O), lambda i: (0, 0)),
            pl.BlockSpec((1, O), lambda i: (0, 0)),
        ],
        out_specs=pl.BlockSpec((br, O), lambda i: (i, 0)),
        out_shape=jax.ShapeDtypeStruct((R, O), jnp.float32),
    )(x, g0.reshape(1, K), b0.reshape(1, K), w, b.reshape(1, O),
      g1.reshape(1, O), b1.reshape(1, O))


def _enc2_body(x_ref, g0_ref, b0_ref, w1_ref, b1_ref, g1_ref, bb1_ref,
               w2_ref, b2_ref, g2_ref, bb2_ref, o_ref):
    x = _ln_in(x_ref[...], g0_ref[...], b0_ref[...])
    y = _dotT(x, w1_ref[...]) + b1_ref[...]
    y = jnp.maximum(_ln_in(y, g1_ref[...], bb1_ref[...]), 0.0)
    z = _dotT(y, w2_ref[...]) + b2_ref[...]
    z = jnp.maximum(_ln_in(z, g2_ref[...], bb2_ref[...]), 0.0)
    o_ref[...] = z


def _enc2(x, g0, b0, w1, b1, g1, bb1, w2, b2, g2, bb2, br=512):
    R, K = x.shape
    M = w1.shape[0]
    O = w2.shape[0]
    grid = (R // br,)
    return pl.pallas_call(
        _enc2_body,
        grid=grid,
        in_specs=[
            pl.BlockSpec((br, K), lambda i: (i, 0)),
            pl.BlockSpec((1, K), lambda i: (0, 0)),
            pl.BlockSpec((1, K), lambda i: (0, 0)),
            pl.BlockSpec((M, K), lambda i: (0, 0)),
            pl.BlockSpec((1, M), lambda i: (0, 0)),
            pl.BlockSpec((1, M), lambda i: (0, 0)),
            pl.BlockSpec((1, M), lambda i: (0, 0)),
            pl.BlockSpec((O, M), lambda i: (0, 0)),
            pl.BlockSpec((1, O), lambda i: (0, 0)),
            pl.BlockSpec((1, O), lambda i: (0, 0)),
            pl.BlockSpec((1, O), lambda i: (0, 0)),
        ],
        out_specs=pl.BlockSpec((br, O), lambda i: (i, 0)),
        out_shape=jax.ShapeDtypeStruct((R, O), jnp.float32),
    )(x, g0.reshape(1, K), b0.reshape(1, K), w1, b1.reshape(1, M),
      g1.reshape(1, M), bb1.reshape(1, M), w2, b2.reshape(1, O),
      g2.reshape(1, O), bb2.reshape(1, O))


def _mm2_body(x_ref, wl_ref, wr_ref, ol_ref, or_ref):
    x = x_ref[...]
    ol_ref[...] = _dotT(x, wl_ref[...])
    or_ref[...] = _dotT(x, wr_ref[...])


def _mm2(x, wl, wr, br=1024):
    R, K = x.shape
    O = wl.shape[0]
    grid = (R // br,)
    return pl.pallas_call(
        _mm2_body,
        grid=grid,
        in_specs=[
            pl.BlockSpec((br, K), lambda i: (i, 0)),
            pl.BlockSpec((O, K), lambda i: (0, 0)),
            pl.BlockSpec((O, K), lambda i: (0, 0)),
        ],
        out_specs=[
            pl.BlockSpec((br, O), lambda i: (i, 0)),
            pl.BlockSpec((br, O), lambda i: (i, 0)),
        ],
        out_shape=[
            jax.ShapeDtypeStruct((R, O), jnp.float32),
            jax.ShapeDtypeStruct((R, O), jnp.float32),
        ],
    )(x, wl, wr)


def _post_body(h_ref, g_ref, lg_ref, lb_ref, o_ref):
    y = _ln_in(g_ref[...], lg_ref[...], lb_ref[...])
    y = jnp.where(y > 0, y, jnp.exp(jnp.minimum(y, 0.0)) - 1.0)
    o_ref[...] = h_ref[...] + y


def _post(h, g, lg, lb, br=1024):
    R, K = h.shape
    grid = (R // br,)
    return pl.pallas_call(
        _post_body,
        grid=grid,
        in_specs=[
            pl.BlockSpec((br, K), lambda i: (i, 0)),
            pl.BlockSpec((br, K), lambda i: (i, 0)),
            pl.BlockSpec((1, K), lambda i: (0, 0)),
            pl.BlockSpec((1, K), lambda i: (0, 0)),
        ],
        out_specs=pl.BlockSpec((br, K), lambda i: (i, 0)),
        out_shape=jax.ShapeDtypeStruct((R, K), jnp.float32),
    )(h, g, lg.reshape(1, K), lb.reshape(1, K))


def _apool_body(B, x_ref, bat_ref, w1_ref, b1_ref, w2_ref, b2_ref, o_ref):
    x = x_ref[...]
    N = x.shape[0]
    t = jnp.tanh(_dotT(x, w1_ref[...]) + b1_ref[...])
    s = jax.lax.dot_general(w2_ref[...], t, (((1,), (1,)), ((), ())),
                            preferred_element_type=jnp.float32)
    s = s + b2_ref[...]
    s = s - jnp.max(s)
    es = jnp.exp(s)
    bat = bat_ref[...]
    oh = (bat == jax.lax.broadcasted_iota(jnp.int32, (B, N), 0)
          ).astype(jnp.float32)
    wm = oh * es
    num = jnp.dot(wm, x, preferred_element_type=jnp.float32)
    den = jnp.sum(wm, axis=1, keepdims=True)
    o_ref[...] = num / (den + 1e-8)


def _apool(x, batch, B, w1, b1, w2, b2):
    N, K = x.shape
    O1 = w1.shape[0]
    return pl.pallas_call(
        functools.partial(_apool_body, B),
        in_specs=[
            pl.BlockSpec((N, K), lambda: (0, 0)),
            pl.BlockSpec((1, N), lambda: (0, 0)),
            pl.BlockSpec((O1, K), lambda: (0, 0)),
            pl.BlockSpec((1, O1), lambda: (0, 0)),
            pl.BlockSpec((1, O1), lambda: (0, 0)),
            pl.BlockSpec((1, 1), lambda: (0, 0)),
        ],
        out_specs=pl.BlockSpec((B, K), lambda: (0, 0)),
        out_shape=jax.ShapeDtypeStruct((B, K), jnp.float32),
    )(x, batch.reshape(1, N).astype(jnp.int32), w1, b1.reshape(1, O1),
      w2, b2.reshape(1, 1))


def _mpool_body(B, x_ref, bat_ref, o_ref):
    x = x_ref[...]
    N = x.shape[0]
    bat = bat_ref[...]
    oh = (bat == jax.lax.broadcasted_iota(jnp.int32, (B, N), 0)
          ).astype(jnp.float32)
    s = jnp.dot(oh, x, preferred_element_type=jnp.float32)
    cnt = jnp.sum(oh, axis=1, keepdims=True)
    o_ref[...] = s / jnp.maximum(cnt, 1.0)


def _mpool(x, batch, B):
    N, K = x.shape
    return pl.pallas_call(
        functools.partial(_mpool_body, B),
        in_specs=[
            pl.BlockSpec((N, K), lambda: (0, 0)),
            pl.BlockSpec((1, N), lambda: (0, 0)),
        ],
        out_specs=pl.BlockSpec((B, K), lambda: (0, 0)),
        out_shape=jax.ShapeDtypeStruct((B, K), jnp.float32),
    )(x, batch.reshape(1, N).astype(jnp.int32))


def _fusion_body(B, nblk, sx_ref, nodes_ref, ab_ref, wq_ref, bq_ref,
                 wk_ref, bk_ref, wv_ref, bv_ref, wo_ref, bo_ref,
                 lg_ref, lb_ref, o_ref, m_ref, l_ref, acc_ref):
    j = pl.program_id(0)

    @pl.when(j == 0)
    def _init():
        m_ref[...] = jnp.full(m_ref.shape, -3e38, jnp.float32)
        l_ref[...] = jnp.zeros(l_ref.shape, jnp.float32)
        acc_ref[...] = jnp.zeros(acc_ref.shape, jnp.float32)

    sx = sx_ref[...]
    q = _dotT(sx, wq_ref[...]) + bq_ref[...]
    nodes = nodes_ref[...]
    k = _dotT(nodes, wk_ref[...]) + bk_ref[...]
    v = _dotT(nodes, wv_ref[...]) + bv_ref[...]
    ab = ab_ref[0]
    BL = nodes.shape[0]
    mask = ab == jax.lax.broadcasted_iota(jnp.int32, (B, BL), 0)
    scale = 1.0 / (C ** 0.5)
    for h in range(H):
        qh = q[:, h * C:(h + 1) * C]
        kh = k[:, h * C:(h + 1) * C]
        vh = v[:, h * C:(h + 1) * C]
        s = jax.lax.dot_general(qh, kh, (((1,), (1,)), ((), ())),
                                preferred_element_type=jnp.float32) * scale
        s = jnp.where(mask, s, -1e30)
        m_old = m_ref[:, h:h + 1]
        m_new = jnp.maximum(m_old, jnp.max(s, axis=1, keepdims=True))
        p = jnp.exp(s - m_new)
        corr = jnp.exp(m_old - m_new)
        l_ref[:, h:h + 1] = l_ref[:, h:h + 1] * corr + jnp.sum(
            p, axis=1, keepdims=True)
        acc_ref[:, h * C:(h + 1) * C] = (
            acc_ref[:, h * C:(h + 1) * C] * corr
            + jnp.dot(p, vh, preferred_element_type=jnp.float32))
        m_ref[:, h:h + 1] = m_new

    @pl.when(j == nblk - 1)
    def _fin():
        acc = acc_ref[...]
        o = jnp.concatenate(
            [acc[:, h * C:(h + 1) * C] / l_ref[:, h:h + 1]
             for h in range(H)], axis=1)
        o = _dotT(o, wo_ref[...]) + bo_ref[...]
        o_ref[...] = _ln_in(sx + o, lg_ref[...], lb_ref[...])


def _fusion(sx, nodes, all_batch, p):
    B = sx.shape[0]
    NT = nodes.shape[0]
    BL = 2048
    nblk = NT // BL
    iw = p['fu_in_w']
    ib = p['fu_in_b']
    ab3 = all_batch.astype(jnp.int32).reshape(nblk, 1, BL)
    return pl.pallas_call(
        functools.partial(_fusion_body, B, nblk),
        grid=(nblk,),
        in_specs=[
            pl.BlockSpec((B, D), lambda j: (0, 0)),
            pl.BlockSpec((BL, D), lambda j: (j, 0)),
            pl.BlockSpec((1, 1, BL), lambda j: (j, 0, 0)),
            pl.BlockSpec((D, D), lambda j: (0, 0)),
            pl.BlockSpec((1, D), lambda j: (0, 0)),
            pl.BlockSpec((D, D), lambda j: (0, 0)),
            pl.BlockSpec((1, D), lambda j: (0, 0)),
            pl.BlockSpec((D, D), lambda j: (0, 0)),
            pl.BlockSpec((1, D), lambda j: (0, 0)),
            pl.BlockSpec((D, D), lambda j: (0, 0)),
            pl.BlockSpec((1, D), lambda j: (0, 0)),
            pl.BlockSpec((1, D), lambda j: (0, 0)),
            pl.BlockSpec((1, D), lambda j: (0, 0)),
        ],
        out_specs=pl.BlockSpec((B, D), lambda j: (0, 0)),
        out_shape=jax.ShapeDtypeStruct((B, D), jnp.float32),
        scratch_shapes=[
            pltpu.VMEM((B, H), jnp.float32),
            pltpu.VMEM((B, H), jnp.float32),
            pltpu.VMEM((B, D), jnp.float32),
        ],
    )(sx, nodes, ab3, iw[:D], ib[:D].reshape(1, D),
      iw[D:2 * D], ib[D:2 * D].reshape(1, D),
      iw[2 * D:], ib[2 * D:].reshape(1, D),
      p['fu_out_w'], p['fu_out_b'].reshape(1, D),
      p['fu_ln_g'].reshape(1, D), p['fu_ln_b'].reshape(1, D))


def _ecf_body(B, hf_ref, fx_ref, bat_ref, fused_ref, al_ref, lg_ref, lb_ref,
              w_ref, b_ref, lam_ref, o_ref):
    bat = bat_ref[0]
    BR = hf_ref.shape[0]
    oh = (bat.T == jax.lax.broadcasted_iota(jnp.int32, (BR, B), 1)
          ).astype(jnp.float32)
    fg = jnp.dot(oh, fused_ref[...], preferred_element_type=jnp.float32)
    t = _ln_in(hf_ref[...] + al_ref[...] * fg, lg_ref[...], lb_ref[...])
    o_ref[...] = t + lam_ref[0, 0] * (_dotT(fx_ref[...], w_ref[...])
                                      + b_ref[...])


def _ecf(hf, fx, batch, fused, p, br=1024):
    R = hf.shape[0]
    B = fused.shape[0]
    nblk = R // br
    bat3 = batch.astype(jnp.int32).reshape(nblk, 1, br)
    return pl.pallas_call(
        functools.partial(_ecf_body, B),
        grid=(nblk,),
        in_specs=[
            pl.BlockSpec((br, D), lambda i: (i, 0)),
            pl.BlockSpec((br, D), lambda i: (i, 0)),
            pl.BlockSpec((1, 1, br), lambda i: (i, 0, 0)),
            pl.BlockSpec((B, D), lambda i: (0, 0)),
            pl.BlockSpec((1, D), lambda i: (0, 0)),
            pl.BlockSpec((1, D), lambda i: (0, 0)),
            pl.BlockSpec((1, D), lambda i: (0, 0)),
            pl.BlockSpec((D, D), lambda i: (0, 0)),
            pl.BlockSpec((1, D), lambda i: (0, 0)),
            pl.BlockSpec((1, 1), lambda i: (0, 0)),
        ],
        out_specs=pl.BlockSpec((br, D), lambda i: (i, 0)),
        out_shape=jax.ShapeDtypeStruct((R, D), jnp.float32),
    )(hf, fx, bat3, fused, p['ec_alpha'].reshape(1, D),
      p['ec_ln_g'].reshape(1, D), p['ec_ln_b'].reshape(1, D),
      p['rfp_W'], p['rfp_b'].reshape(1, D),
      p['lambda_face'].reshape(1, 1))


def _resl_body(h_ref, x_ref, w_ref, b_ref, lam_ref, o_ref):
    o_ref[...] = h_ref[...] + lam_ref[0, 0] * (
        _dotT(x_ref[...], w_ref[...]) + b_ref[...])


def _resl(h, x, w, b, lam, br=1024):
    R = h.shape[0]
    return pl.pallas_call(
        _resl_body,
        grid=(R // br,),
        in_specs=[
            pl.BlockSpec((br, D), lambda i: (i, 0)),
            pl.BlockSpec((br, D), lambda i: (i, 0)),
            pl.BlockSpec((D, D), lambda i: (0, 0)),
            pl.BlockSpec((1, D), lambda i: (0, 0)),
            pl.BlockSpec((1, 1), lambda i: (0, 0)),
        ],
        out_specs=pl.BlockSpec((br, D), lambda i: (i, 0)),
        out_shape=jax.ShapeDtypeStruct((R, D), jnp.float32),
    )(h, x, w, b.reshape(1, D), lam.reshape(1, 1))


def _heads_body(pf_ref, mo_ref, sx_ref, ff_ref, fo_ref, fu_ref,
                cf_w, cf_b, cc_w, cc_b, cs_w, cs_b,
                w1_ref, b1_ref, lg_ref, lb_ref, w2_ref, b2_ref,
                out_ref, of_ref, oc_ref, os_ref):
    of_ref[...] = _dotT(pf_ref[...], cf_w[...]) + cf_b[...]
    oc_ref[...] = _dotT(mo_ref[...], cc_w[...]) + cc_b[...]
    os_ref[...] = _dotT(sx_ref[...], cs_w[...]) + cs_b[...]
    comb = jnp.concatenate([ff_ref[...], fo_ref[...], fu_ref[...]], axis=1)
    h = _dotT(comb, w1_ref[...]) + b1_ref[...]
    h = jnp.maximum(_ln_in(h, lg_ref[...], lb_ref[...]), 0.0)
    out_ref[...] = _dotT(h, w2_ref[...]) + b2_ref[...]


def _heads(pf, mo, sx, ff, fo, fu, p):
    B = pf.shape[0]
    full = lambda a: pl.BlockSpec(a.shape, lambda: tuple(0 for _ in a.shape))
    args = [pf, mo, sx, ff, fo, fu,
            p['cf_W'], p['cf_b'].reshape(1, 3),
            p['cc_W'], p['cc_b'].reshape(1, 3),
            p['cs_W'], p['cs_b'].reshape(1, 3),
            p['cw_W1'], p['cw_b1'].reshape(1, D),
            p['cw_ln_g'].reshape(1, D), p['cw_ln_b'].reshape(1, D),
            p['cw_W2'], p['cw_b2'].reshape(1, 3)]
    return pl.pallas_call(
        _heads_body,
        in_specs=[full(a) for a in args],
        out_specs=[pl.BlockSpec((B, 3), lambda: (0, 0))] * 4,
        out_shape=[jax.ShapeDtypeStruct((B, 3), jnp.float32)] * 4,
    )(*args)


def _edge_phase(xl, xr, src, dst, att, N):
    xlr = xl.reshape(-1, H, C)
    e = xlr[src] + xr.reshape(-1, H, C)[dst]
    e = jnp.where(e > 0, e, 0.2 * e)
    score = jnp.sum(e * att[None, :, :], axis=-1)
    m = jax.ops.segment_max(score, dst, num_segments=N)
    m = jnp.where(jnp.isfinite(m), m, 0.0)
    ex = jnp.exp(score - m[dst])
    den = jax.ops.segment_sum(ex, dst, num_segments=N)
    alpha = ex / (den[dst] + 1e-16)
    out = jax.ops.segment_sum(alpha[:, :, None] * xlr[src], dst,
                              num_segments=N)
    return out.reshape(N, D)


def _mlgat(x, edge_index, p, pre):
    N = x.shape[0]
    loops = jnp.arange(N, dtype=edge_index.dtype)
    src = jnp.concatenate([edge_index[0], loops])
    dst = jnp.concatenate([edge_index[1], loops])
    h = _enc1(x, jnp.ones((D,), jnp.float32), jnp.zeros((D,), jnp.float32),
              p[pre + '_inW'], p[pre + '_inb'],
              p[pre + '_inln_g'], p[pre + '_inln_b'], pre_ln=False)
    for l in range(2):
        xl, xr = _mm2(h, p[pre + '_Wl%d' % l], p[pre + '_Wr%d' % l])
        g = _edge_phase(xl, xr, src, dst, p[pre + '_att%d' % l], N)
        h = _post(h, g, p[pre + '_ln%d_g' % l], p[pre + '_ln%d_b' % l])
    return h


def kernel(face_x, context_x, scene_x, params, face_edge_index,
           context_edge_index, face_batch, context_batch):
    p = params
    B = scene_x.shape[0]

    fx = _enc2(face_x, p['rf_ln0_g'], p['rf_ln0_b'],
               p['rf_W1'], p['rf_b1'], p['rf_ln1_g'], p['rf_ln1_b'],
               p['rf_W2'], p['rf_b2'], p['rf_ln2_g'], p['rf_ln2_b'])
    ox = _enc1(context_x, p['ro_ln0_g'], p['ro_ln0_b'],
               p['ro_W'], p['ro_b'], p['ro_ln1_g'], p['ro_ln1_b'])
    sx = _enc1(scene_x, p['rs_ln0_g'], p['rs_ln0_b'],
               p['rs_W'], p['rs_b'], p['rs_ln1_g'], p['rs_ln1_b'], br=64)

    Hf = _mlgat(fx, face_edge_index, p, 'fg')
    Ho = _mlgat(ox, context_edge_index, p, 'cg')

    pf0 = _apool(Hf, face_batch, B, p['apb_W1'], p['apb_b1'],
                 p['apb_W2'], p['apb_b2'])
    mo0 = _mpool(Ho, context_batch, B)

    nodes = jnp.concatenate([Hf, Ho], axis=0)
    all_batch = jnp.concatenate([face_batch, context_batch])
    fused = _fusion(sx, nodes, all_batch, p)

    Hf2 = _ecf(Hf, fx, face_batch, fused, p)
    Ho2 = _resl(Ho, ox, p['rop_W'], p['rop_b'], p['lambda_obj'])

    ff = _apool(Hf2, face_batch, B, p['apf_W1'], p['apf_b1'],
                p['apf_W2'], p['apf_b2'])
    fo = _mpool(Ho2, context_batch, B)

    out, out_face, out_context, out_scene = _heads(
        pf0, mo0, sx, ff, fo, fused, p)
    return out, out_face, out_context, out_scene


# speedup vs baseline: 5.5212x; 5.0636x over previous
import functools
import jax
import jax.numpy as jnp
from jax.experimental import pallas as pl
from jax.experimental.pallas import tpu as pltpu

D = 512
H = 4
C = 128
_EPS = 1e-5


def _ln_in(x, g, b):
    m = jnp.mean(x, axis=-1, keepdims=True)
    v = jnp.mean((x - m) ** 2, axis=-1, keepdims=True)
    return (x - m) * jax.lax.rsqrt(v + _EPS) * g + b


def _dotT(x, w):
    return jax.lax.dot_general(x, w, (((1,), (1,)), ((), ())),
                               preferred_element_type=jnp.float32)


def _enc1_body(pre_ln, x_ref, g0_ref, b0_ref, w_ref, b_ref, g1_ref, b1_ref,
               o_ref):
    x = x_ref[...]
    if pre_ln:
        x = _ln_in(x, g0_ref[...], b0_ref[...])
    y = _dotT(x, w_ref[...]) + b_ref[...]
    y = _ln_in(y, g1_ref[...], b1_ref[...])
    o_ref[...] = jnp.maximum(y, 0.0)


def _enc1(x, g0, b0, w, b, g1, b1, pre_ln=True, br=512):
    R, K = x.shape
    O = w.shape[0]
    if R < br:
        br = R
    grid = (R // br,)
    return pl.pallas_call(
        functools.partial(_enc1_body, pre_ln),
        grid=grid,
        in_specs=[
            pl.BlockSpec((br, K), lambda i: (i, 0)),
            pl.BlockSpec((1, K), lambda i: (0, 0)),
            pl.BlockSpec((1, K), lambda i: (0, 0)),
            pl.BlockSpec((O, K), lambda i: (0, 0)),
            pl.BlockSpec((1, O), lambda i: (0, 0)),
            pl.BlockSpec((1, O), lambda i: (0, 0)),
            pl.BlockSpec((1, O), lambda i: (0, 0)),
        ],
        out_specs=pl.BlockSpec((br, O), lambda i: (i, 0)),
        out_shape=jax.ShapeDtypeStruct((R, O), jnp.float32),
    )(x, g0.reshape(1, K), b0.reshape(1, K), w, b.reshape(1, O),
      g1.reshape(1, O), b1.reshape(1, O))


def _enc2_body(x_ref, g0_ref, b0_ref, w1_ref, b1_ref, g1_ref, bb1_ref,
               w2_ref, b2_ref, g2_ref, bb2_ref, o_ref):
    x = _ln_in(x_ref[...], g0_ref[...], b0_ref[...])
    y = _dotT(x, w1_ref[...]) + b1_ref[...]
    y = jnp.maximum(_ln_in(y, g1_ref[...], bb1_ref[...]), 0.0)
    z = _dotT(y, w2_ref[...]) + b2_ref[...]
    z = jnp.maximum(_ln_in(z, g2_ref[...], bb2_ref[...]), 0.0)
    o_ref[...] = z


def _enc2(x, g0, b0, w1, b1, g1, bb1, w2, b2, g2, bb2, br=512):
    R, K = x.shape
    M = w1.shape[0]
    O = w2.shape[0]
    grid = (R // br,)
    return pl.pallas_call(
        _enc2_body,
        grid=grid,
        in_specs=[
            pl.BlockSpec((br, K), lambda i: (i, 0)),
            pl.BlockSpec((1, K), lambda i: (0, 0)),
            pl.BlockSpec((1, K), lambda i: (0, 0)),
            pl.BlockSpec((M, K), lambda i: (0, 0)),
            pl.BlockSpec((1, M), lambda i: (0, 0)),
            pl.BlockSpec((1, M), lambda i: (0, 0)),
            pl.BlockSpec((1, M), lambda i: (0, 0)),
            pl.BlockSpec((O, M), lambda i: (0, 0)),
            pl.BlockSpec((1, O), lambda i: (0, 0)),
            pl.BlockSpec((1, O), lambda i: (0, 0)),
            pl.BlockSpec((1, O), lambda i: (0, 0)),
        ],
        out_specs=pl.BlockSpec((br, O), lambda i: (i, 0)),
        out_shape=jax.ShapeDtypeStruct((R, O), jnp.float32),
    )(x, g0.reshape(1, K), b0.reshape(1, K), w1, b1.reshape(1, M),
      g1.reshape(1, M), bb1.reshape(1, M), w2, b2.reshape(1, O),
      g2.reshape(1, O), bb2.reshape(1, O))


def _mm2_body(x_ref, wl_ref, wr_ref, ol_ref, or_ref):
    x = x_ref[...]
    ol_ref[...] = _dotT(x, wl_ref[...])
    or_ref[...] = _dotT(x, wr_ref[...])


def _mm2(x, wl, wr, br=1024):
    R, K = x.shape
    O = wl.shape[0]
    grid = (R // br,)
    return pl.pallas_call(
        _mm2_body,
        grid=grid,
        in_specs=[
            pl.BlockSpec((br, K), lambda i: (i, 0)),
            pl.BlockSpec((O, K), lambda i: (0, 0)),
            pl.BlockSpec((O, K), lambda i: (0, 0)),
        ],
        out_specs=[
            pl.BlockSpec((br, O), lambda i: (i, 0)),
            pl.BlockSpec((br, O), lambda i: (i, 0)),
        ],
        out_shape=[
            jax.ShapeDtypeStruct((R, O), jnp.float32),
            jax.ShapeDtypeStruct((R, O), jnp.float32),
        ],
    )(x, wl, wr)


def _post_body(h_ref, g_ref, lg_ref, lb_ref, o_ref):
    y = _ln_in(g_ref[...], lg_ref[...], lb_ref[...])
    y = jnp.where(y > 0, y, jnp.exp(jnp.minimum(y, 0.0)) - 1.0)
    o_ref[...] = h_ref[...] + y


def _post(h, g, lg, lb, br=1024):
    R, K = h.shape
    grid = (R // br,)
    return pl.pallas_call(
        _post_body,
        grid=grid,
        in_specs=[
            pl.BlockSpec((br, K), lambda i: (i, 0)),
            pl.BlockSpec((br, K), lambda i: (i, 0)),
            pl.BlockSpec((1, K), lambda i: (0, 0)),
            pl.BlockSpec((1, K), lambda i: (0, 0)),
        ],
        out_specs=pl.BlockSpec((br, K), lambda i: (i, 0)),
        out_shape=jax.ShapeDtypeStruct((R, K), jnp.float32),
    )(h, g, lg.reshape(1, K), lb.reshape(1, K))


def _apool_body(B, x_ref, bat_ref, w1_ref, b1_ref, w2_ref, b2_ref, o_ref):
    x = x_ref[...]
    N = x.shape[0]
    t = jnp.tanh(_dotT(x, w1_ref[...]) + b1_ref[...])
    s = jax.lax.dot_general(w2_ref[...], t, (((1,), (1,)), ((), ())),
                            preferred_element_type=jnp.float32)
    s = s + b2_ref[...]
    s = s - jnp.max(s)
    es = jnp.exp(s)
    bat = bat_ref[...]
    oh = (bat == jax.lax.broadcasted_iota(jnp.int32, (B, N), 0)
          ).astype(jnp.float32)
    wm = oh * es
    num = jnp.dot(wm, x, preferred_element_type=jnp.float32)
    den = jnp.sum(wm, axis=1, keepdims=True)
    o_ref[...] = num / (den + 1e-8)


def _apool(x, batch, B, w1, b1, w2, b2):
    N, K = x.shape
    O1 = w1.shape[0]
    return pl.pallas_call(
        functools.partial(_apool_body, B),
        in_specs=[
            pl.BlockSpec((N, K), lambda: (0, 0)),
            pl.BlockSpec((1, N), lambda: (0, 0)),
            pl.BlockSpec((O1, K), lambda: (0, 0)),
            pl.BlockSpec((1, O1), lambda: (0, 0)),
            pl.BlockSpec((1, O1), lambda: (0, 0)),
            pl.BlockSpec((1, 1), lambda: (0, 0)),
        ],
        out_specs=pl.BlockSpec((B, K), lambda: (0, 0)),
        out_shape=jax.ShapeDtypeStruct((B, K), jnp.float32),
    )(x, batch.reshape(1, N).astype(jnp.int32), w1, b1.reshape(1, O1),
      w2, b2.reshape(1, 1))


def _mpool_body(B, x_ref, bat_ref, o_ref):
    x = x_ref[...]
    N = x.shape[0]
    bat = bat_ref[...]
    oh = (bat == jax.lax.broadcasted_iota(jnp.int32, (B, N), 0)
          ).astype(jnp.float32)
    s = jnp.dot(oh, x, preferred_element_type=jnp.float32)
    cnt = jnp.sum(oh, axis=1, keepdims=True)
    o_ref[...] = s / jnp.maximum(cnt, 1.0)


def _mpool(x, batch, B):
    N, K = x.shape
    return pl.pallas_call(
        functools.partial(_mpool_body, B),
        in_specs=[
            pl.BlockSpec((N, K), lambda: (0, 0)),
            pl.BlockSpec((1, N), lambda: (0, 0)),
        ],
        out_specs=pl.BlockSpec((B, K), lambda: (0, 0)),
        out_shape=jax.ShapeDtypeStruct((B, K), jnp.float32),
    )(x, batch.reshape(1, N).astype(jnp.int32))


def _fusion_body(B, nblk, sx_ref, nodes_ref, ab_ref, wq_ref, bq_ref,
                 wk_ref, bk_ref, wv_ref, bv_ref, wo_ref, bo_ref,
                 lg_ref, lb_ref, o_ref, m_ref, l_ref, acc_ref):
    j = pl.program_id(0)

    @pl.when(j == 0)
    def _init():
        m_ref[...] = jnp.full(m_ref.shape, -3e38, jnp.float32)
        l_ref[...] = jnp.zeros(l_ref.shape, jnp.float32)
        acc_ref[...] = jnp.zeros(acc_ref.shape, jnp.float32)

    sx = sx_ref[...]
    q = _dotT(sx, wq_ref[...]) + bq_ref[...]
    nodes = nodes_ref[...]
    k = _dotT(nodes, wk_ref[...]) + bk_ref[...]
    v = _dotT(nodes, wv_ref[...]) + bv_ref[...]
    ab = ab_ref[0]
    BL = nodes.shape[0]
    mask = ab == jax.lax.broadcasted_iota(jnp.int32, (B, BL), 0)
    scale = 1.0 / (C ** 0.5)
    for h in range(H):
        qh = q[:, h * C:(h + 1) * C]
        kh = k[:, h * C:(h + 1) * C]
        vh = v[:, h * C:(h + 1) * C]
        s = jax.lax.dot_general(qh, kh, (((1,), (1,)), ((), ())),
                                preferred_element_type=jnp.float32) * scale
        s = jnp.where(mask, s, -1e30)
        m_old = m_ref[:, h:h + 1]
        m_new = jnp.maximum(m_old, jnp.max(s, axis=1, keepdims=True))
        p = jnp.exp(s - m_new)
        corr = jnp.exp(m_old - m_new)
        l_ref[:, h:h + 1] = l_ref[:, h:h + 1] * corr + jnp.sum(
            p, axis=1, keepdims=True)
        acc_ref[:, h * C:(h + 1) * C] = (
            acc_ref[:, h * C:(h + 1) * C] * corr
            + jnp.dot(p, vh, preferred_element_type=jnp.float32))
        m_ref[:, h:h + 1] = m_new

    @pl.when(j == nblk - 1)
    def _fin():
        acc = acc_ref[...]
        o = jnp.concatenate(
            [acc[:, h * C:(h + 1) * C] / l_ref[:, h:h + 1]
             for h in range(H)], axis=1)
        o = _dotT(o, wo_ref[...]) + bo_ref[...]
        o_ref[...] = _ln_in(sx + o, lg_ref[...], lb_ref[...])


def _fusion(sx, nodes, all_batch, p):
    B = sx.shape[0]
    NT = nodes.shape[0]
    BL = 2048
    nblk = NT // BL
    iw = p['fu_in_w']
    ib = p['fu_in_b']
    ab3 = all_batch.astype(jnp.int32).reshape(nblk, 1, BL)
    return pl.pallas_call(
        functools.partial(_fusion_body, B, nblk),
        grid=(nblk,),
        in_specs=[
            pl.BlockSpec((B, D), lambda j: (0, 0)),
            pl.BlockSpec((BL, D), lambda j: (j, 0)),
            pl.BlockSpec((1, 1, BL), lambda j: (j, 0, 0)),
            pl.BlockSpec((D, D), lambda j: (0, 0)),
            pl.BlockSpec((1, D), lambda j: (0, 0)),
            pl.BlockSpec((D, D), lambda j: (0, 0)),
            pl.BlockSpec((1, D), lambda j: (0, 0)),
            pl.BlockSpec((D, D), lambda j: (0, 0)),
            pl.BlockSpec((1, D), lambda j: (0, 0)),
            pl.BlockSpec((D, D), lambda j: (0, 0)),
            pl.BlockSpec((1, D), lambda j: (0, 0)),
            pl.BlockSpec((1, D), lambda j: (0, 0)),
            pl.BlockSpec((1, D), lambda j: (0, 0)),
        ],
        out_specs=pl.BlockSpec((B, D), lambda j: (0, 0)),
        out_shape=jax.ShapeDtypeStruct((B, D), jnp.float32),
        scratch_shapes=[
            pltpu.VMEM((B, H), jnp.float32),
            pltpu.VMEM((B, H), jnp.float32),
            pltpu.VMEM((B, D), jnp.float32),
        ],
    )(sx, nodes, ab3, iw[:D], ib[:D].reshape(1, D),
      iw[D:2 * D], ib[D:2 * D].reshape(1, D),
      iw[2 * D:], ib[2 * D:].reshape(1, D),
      p['fu_out_w'], p['fu_out_b'].reshape(1, D),
      p['fu_ln_g'].reshape(1, D), p['fu_ln_b'].reshape(1, D))


def _ecf_body(B, hf_ref, fx_ref, bat_ref, fused_ref, al_ref, lg_ref, lb_ref,
              w_ref, b_ref, lam_ref, o_ref):
    bat = bat_ref[0]
    BR = hf_ref.shape[0]
    oh = (bat.T == jax.lax.broadcasted_iota(jnp.int32, (BR, B), 1)
          ).astype(jnp.float32)
    fg = jnp.dot(oh, fused_ref[...], preferred_element_type=jnp.float32)
    t = _ln_in(hf_ref[...] + al_ref[...] * fg, lg_ref[...], lb_ref[...])
    o_ref[...] = t + lam_ref[0, 0] * (_dotT(fx_ref[...], w_ref[...])
                                      + b_ref[...])


def _ecf(hf, fx, batch, fused, p, br=1024):
    R = hf.shape[0]
    B = fused.shape[0]
    nblk = R // br
    bat3 = batch.astype(jnp.int32).reshape(nblk, 1, br)
    return pl.pallas_call(
        functools.partial(_ecf_body, B),
        grid=(nblk,),
        in_specs=[
            pl.BlockSpec((br, D), lambda i: (i, 0)),
            pl.BlockSpec((br, D), lambda i: (i, 0)),
            pl.BlockSpec((1, 1, br), lambda i: (i, 0, 0)),
            pl.BlockSpec((B, D), lambda i: (0, 0)),
            pl.BlockSpec((1, D), lambda i: (0, 0)),
            pl.BlockSpec((1, D), lambda i: (0, 0)),
            pl.BlockSpec((1, D), lambda i: (0, 0)),
            pl.BlockSpec((D, D), lambda i: (0, 0)),
            pl.BlockSpec((1, D), lambda i: (0, 0)),
            pl.BlockSpec((1, 1), lambda i: (0, 0)),
        ],
        out_specs=pl.BlockSpec((br, D), lambda i: (i, 0)),
        out_shape=jax.ShapeDtypeStruct((R, D), jnp.float32),
    )(hf, fx, bat3, fused, p['ec_alpha'].reshape(1, D),
      p['ec_ln_g'].reshape(1, D), p['ec_ln_b'].reshape(1, D),
      p['rfp_W'], p['rfp_b'].reshape(1, D),
      p['lambda_face'].reshape(1, 1))


def _resl_body(h_ref, x_ref, w_ref, b_ref, lam_ref, o_ref):
    o_ref[...] = h_ref[...] + lam_ref[0, 0] * (
        _dotT(x_ref[...], w_ref[...]) + b_ref[...])


def _resl(h, x, w, b, lam, br=1024):
    R = h.shape[0]
    return pl.pallas_call(
        _resl_body,
        grid=(R // br,),
        in_specs=[
            pl.BlockSpec((br, D), lambda i: (i, 0)),
            pl.BlockSpec((br, D), lambda i: (i, 0)),
            pl.BlockSpec((D, D), lambda i: (0, 0)),
            pl.BlockSpec((1, D), lambda i: (0, 0)),
            pl.BlockSpec((1, 1), lambda i: (0, 0)),
        ],
        out_specs=pl.BlockSpec((br, D), lambda i: (i, 0)),
        out_shape=jax.ShapeDtypeStruct((R, D), jnp.float32),
    )(h, x, w, b.reshape(1, D), lam.reshape(1, 1))


def _heads_body(pf_ref, mo_ref, sx_ref, ff_ref, fo_ref, fu_ref,
                cf_w, cf_b, cc_w, cc_b, cs_w, cs_b,
                w1_ref, b1_ref, lg_ref, lb_ref, w2_ref, b2_ref,
                out_ref, of_ref, oc_ref, os_ref):
    of_ref[...] = _dotT(pf_ref[...], cf_w[...]) + cf_b[...]
    oc_ref[...] = _dotT(mo_ref[...], cc_w[...]) + cc_b[...]
    os_ref[...] = _dotT(sx_ref[...], cs_w[...]) + cs_b[...]
    comb = jnp.concatenate([ff_ref[...], fo_ref[...], fu_ref[...]], axis=1)
    h = _dotT(comb, w1_ref[...]) + b1_ref[...]
    h = jnp.maximum(_ln_in(h, lg_ref[...], lb_ref[...]), 0.0)
    out_ref[...] = _dotT(h, w2_ref[...]) + b2_ref[...]


def _heads(pf, mo, sx, ff, fo, fu, p):
    B = pf.shape[0]
    full = lambda a: pl.BlockSpec(a.shape, lambda: tuple(0 for _ in a.shape))
    args = [pf, mo, sx, ff, fo, fu,
            p['cf_W'], p['cf_b'].reshape(1, 3),
            p['cc_W'], p['cc_b'].reshape(1, 3),
            p['cs_W'], p['cs_b'].reshape(1, 3),
            p['cw_W1'], p['cw_b1'].reshape(1, D),
            p['cw_ln_g'].reshape(1, D), p['cw_ln_b'].reshape(1, D),
            p['cw_W2'], p['cw_b2'].reshape(1, 3)]
    return pl.pallas_call(
        _heads_body,
        in_specs=[full(a) for a in args],
        out_specs=[pl.BlockSpec((B, 3), lambda: (0, 0))] * 4,
        out_shape=[jax.ShapeDtypeStruct((B, 3), jnp.float32)] * 4,
    )(*args)


def _edge_phase(xl, xr, src, dst, att, N):
    E = src.shape[0]
    table = jnp.concatenate([xl, xr], axis=0)
    g = table[jnp.concatenate([src, dst + N])].reshape(2, E, H, C)
    e = g[0] + g[1]
    e = jnp.where(e > 0, e, 0.2 * e)
    score = jnp.sum(e * att[None, :, :], axis=-1)
    ex = jnp.exp(score - jnp.max(score))
    msg = jnp.concatenate(
        [(ex[:, :, None] * g[0]).reshape(E, D), ex], axis=1)
    acc = jax.ops.segment_sum(msg, dst, num_segments=N,
                              indices_are_sorted=True)
    den = acc[:, D:]
    out = acc[:, :D].reshape(N, H, C) / (den[:, :, None] + 1e-16)
    return out.reshape(N, D)


def _mlgat(x, edge_index, p, pre):
    N = x.shape[0]
    loops = jnp.arange(N, dtype=edge_index.dtype)
    src = jnp.concatenate([edge_index[0], loops])
    dst = jnp.concatenate([edge_index[1], loops])
    order = jnp.argsort(dst)
    src = src[order]
    dst = dst[order]
    h = _enc1(x, jnp.ones((D,), jnp.float32), jnp.zeros((D,), jnp.float32),
              p[pre + '_inW'], p[pre + '_inb'],
              p[pre + '_inln_g'], p[pre + '_inln_b'], pre_ln=False)
    for l in range(2):
        xl, xr = _mm2(h, p[pre + '_Wl%d' % l], p[pre + '_Wr%d' % l])
        g = _edge_phase(xl, xr, src, dst, p[pre + '_att%d' % l], N)
        h = _post(h, g, p[pre + '_ln%d_g' % l], p[pre + '_ln%d_b' % l])
    return h


def kernel(face_x, context_x, scene_x, params, face_edge_index,
           context_edge_index, face_batch, context_batch):
    p = params
    B = scene_x.shape[0]

    fx = _enc2(face_x, p['rf_ln0_g'], p['rf_ln0_b'],
               p['rf_W1'], p['rf_b1'], p['rf_ln1_g'], p['rf_ln1_b'],
               p['rf_W2'], p['rf_b2'], p['rf_ln2_g'], p['rf_ln2_b'])
    ox = _enc1(context_x, p['ro_ln0_g'], p['ro_ln0_b'],
               p['ro_W'], p['ro_b'], p['ro_ln1_g'], p['ro_ln1_b'])
    sx = _enc1(scene_x, p['rs_ln0_g'], p['rs_ln0_b'],
               p['rs_W'], p['rs_b'], p['rs_ln1_g'], p['rs_ln1_b'], br=64)

    Hf = _mlgat(fx, face_edge_index, p, 'fg')
    Ho = _mlgat(ox, context_edge_index, p, 'cg')

    pf0 = _apool(Hf, face_batch, B, p['apb_W1'], p['apb_b1'],
                 p['apb_W2'], p['apb_b2'])
    mo0 = _mpool(Ho, context_batch, B)

    nodes = jnp.concatenate([Hf, Ho], axis=0)
    all_batch = jnp.concatenate([face_batch, context_batch])
    fused = _fusion(sx, nodes, all_batch, p)

    Hf2 = _ecf(Hf, fx, face_batch, fused, p)
    Ho2 = _resl(Ho, ox, p['rop_W'], p['rop_b'], p['lambda_obj'])

    ff = _apool(Hf2, face_batch, B, p['apf_W1'], p['apf_b1'],
                p['apf_W2'], p['apf_b2'])
    fo = _mpool(Ho2, context_batch, B)

    out, out_face, out_context, out_scene = _heads(
        pf0, mo0, sx, ff, fo, fused, p)
    return out, out_face, out_context, out_scene
